# baseline (speedup 1.0000x reference)
import functools

import jax
import jax.numpy as jnp
from jax import lax
from jax.experimental import pallas as pl
from jax.experimental.pallas import tpu as pltpu
from jax.experimental.pallas import tpu_sc as plsc

N = 10000
E = 320000
D = 128
ED = 16
H = 128

NC = 2
NS = 16
NW = NC * NS
C = 64
NCHUNK = E // C
KMAX = -(-NCHUNK // NW)
KP = ((KMAX + 2) // 3) * 3
RPT = 640
RPT_LAST = N - 15 * RPT


def _sc_message_pass(h, e, src2, dst2):
    mesh = plsc.VectorSubcoreMesh(core_axis_name="c", subcore_axis_name="s")

    @functools.partial(
        pl.kernel,
        mesh=mesh,
        out_type=jax.ShapeDtypeStruct((NC, N, D), jnp.float32),
        scratch_types=(
            [pltpu.VMEM((C,), jnp.int32)] * 3
            + [pltpu.VMEM((C,), jnp.int32)] * 3
            + [pltpu.VMEM((C, D), jnp.float32)] * 3
            + [pltpu.VMEM((C, D), jnp.float32)] * 3
            + [pltpu.VMEM_SHARED((N, D), jnp.float32)]
            + [pltpu.SemaphoreType.DMA] * 6
        ),
    )
    def body(h_hbm, e_hbm, src_hbm, dst_hbm, out_hbm,
             src0, src1, src2v, dst0, dst1, dst2v, rows0, rows1, rows2,
             ev0, ev1, ev2, acc, sg0, sg1, sg2, ss0, ss1, ss2):
        bufs = ((src0, dst0, rows0, ev0, sg0, ss0),
                (src1, dst1, rows1, ev1, sg1, ss1),
                (src2v, dst2v, rows2, ev2, sg2, ss2))
        cid = lax.axis_index("c")
        sid = lax.axis_index("s")
        wid = cid * NS + sid

        r0 = sid * RPT

        @pl.when(sid < NS - 1)
        def _():
            pltpu.sync_copy(h_hbm.at[pl.ds(r0, RPT)], acc.at[pl.ds(r0, RPT)])

        @pl.when(sid == NS - 1)
        def _():
            pltpu.sync_copy(h_hbm.at[pl.ds(r0, RPT_LAST)],
                            acc.at[pl.ds(r0, RPT_LAST)])

        plsc.subcore_barrier()

        def start(k, b):
            sv, dv, rv, ev, sg, ss = bufs[b]
            chunk = wid + NW * k

            @pl.when(chunk < NCHUNK)
            def _():
                base = chunk * C
                pltpu.sync_copy(src_hbm.at[pl.ds(base, C)], sv)
                pltpu.sync_copy(dst_hbm.at[pl.ds(base, C)], dv)
                pltpu.async_copy(h_hbm.at[sv], rv, sg)
                pltpu.async_copy(e_hbm.at[pl.ds(base, C)], ev, sg)

        def finish(k, b):
            sv, dv, rv, ev, sg, ss = bufs[b]
            chunk = wid + NW * k

            @pl.when(chunk < NCHUNK)
            def _():
                base = chunk * C
                pltpu.make_async_copy(h_hbm.at[sv], rv, sg).wait()
                pltpu.make_async_copy(e_hbm.at[pl.ds(base, C)], ev,
                                      sg).wait()

                def crow(r, c2):
                    for t in range(2):
                        for j in range(D // 16):
                            s = pl.ds(j * 16, 16)
                            rv[2 * r + t, s] = jnp.maximum(
                                rv[2 * r + t, s] + ev[2 * r + t, s], 0.0)
                    return c2

                lax.fori_loop(0, C // 2, crow, 0)
                pltpu.sync_copy(rv, acc.at[dv], add=True)

        start(0, 0)
        start(1, 1)

        def step(k3, carry):
            k = 3 * k3
            for t in range(3):
                start(k + t + 2, (t + 2) % 3)
                finish(k + t, t)
            return carry

        lax.fori_loop(0, KP // 3, step, 0)
        plsc.subcore_barrier()

        @pl.when(sid < NS - 1)
        def _():
            pltpu.sync_copy(acc.at[pl.ds(r0, RPT)],
                            out_hbm.at[cid, pl.ds(r0, RPT)])

        @pl.when(sid == NS - 1)
        def _():
            pltpu.sync_copy(acc.at[pl.ds(r0, RPT_LAST)],
                            out_hbm.at[cid, pl.ds(r0, RPT_LAST)])

    return body(h, e, src2, dst2)


BE = 4000


def _edge_linear(edge_attr, W_stack, b_stack):
    def body(ea_ref, w_ref, b_ref, o0, o1, o2):
        ea = ea_ref[...]
        outs = (o0, o1, o2)
        for l in range(3):
            outs[l][...] = jnp.dot(ea, w_ref[l],
                                   preferred_element_type=jnp.float32) + b_ref[l]

    return pl.pallas_call(
        body,
        grid=(E // BE,),
        in_specs=[
            pl.BlockSpec((BE, ED), lambda i: (i, 0)),
            pl.BlockSpec((3, ED, D), lambda i: (0, 0, 0)),
            pl.BlockSpec((3, D), lambda i: (0, 0)),
        ],
        out_specs=[pl.BlockSpec((BE, D), lambda i: (i, 0))] * 3,
        out_shape=[jax.ShapeDtypeStruct((E, D), jnp.float32)] * 3,
    )(edge_attr, W_stack, b_stack)


BN = 2000


def _mlp(h, agg, W1, b1, W2, b2):
    def body(h_ref, a_ref, w1, b1r, w2, b2r, out_ref):
        z = a_ref[0] + a_ref[1] - h_ref[...]
        z1 = jnp.maximum(jnp.dot(z, w1[...],
                                 preferred_element_type=jnp.float32) + b1r[...], 0.0)
        z2 = jnp.dot(z1, w2[...], preferred_element_type=jnp.float32) + b2r[...]
        out_ref[...] = jnp.maximum(z2, 0.0)

    return pl.pallas_call(
        body,
        grid=(N // BN,),
        in_specs=[
            pl.BlockSpec((BN, D), lambda i: (i, 0)),
            pl.BlockSpec((NC, BN, D), lambda i: (0, i, 0)),
            pl.BlockSpec((D, H), lambda i: (0, 0)),
            pl.BlockSpec((H,), lambda i: (0,)),
            pl.BlockSpec((H, H), lambda i: (0, 0)),
            pl.BlockSpec((H,), lambda i: (0,)),
        ],
        out_specs=pl.BlockSpec((BN, H), lambda i: (i, 0)),
        out_shape=jax.ShapeDtypeStruct((N, H), jnp.float32),
    )(h, agg, W1, b1, W2, b2)


def _mlp_head(h, agg, W1, b1, W2, b2, Wout, bout):
    def body(h_ref, a_ref, w1, b1r, w2, b2r, wo, bo, out_ref):
        z = a_ref[0] + a_ref[1] - h_ref[...]
        z1 = jnp.maximum(jnp.dot(z, w1[...],
                                 preferred_element_type=jnp.float32) + b1r[...], 0.0)
        z2 = jnp.dot(z1, w2[...], preferred_element_type=jnp.float32) + b2r[...]
        hf = jnp.maximum(z2, 0.0)
        logit = jnp.dot(hf, wo[...], preferred_element_type=jnp.float32) + bo[...]
        out_ref[...] = jax.nn.sigmoid(logit)

    return pl.pallas_call(
        body,
        grid=(N // BN,),
        in_specs=[
            pl.BlockSpec((BN, D), lambda i: (i, 0)),
            pl.BlockSpec((NC, BN, D), lambda i: (0, i, 0)),
            pl.BlockSpec((D, H), lambda i: (0, 0)),
            pl.BlockSpec((H,), lambda i: (0,)),
            pl.BlockSpec((H, H), lambda i: (0, 0)),
            pl.BlockSpec((H,), lambda i: (0,)),
            pl.BlockSpec((H, 1), lambda i: (0, 0)),
            pl.BlockSpec((1,), lambda i: (0,)),
        ],
        out_specs=pl.BlockSpec((BN, 1), lambda i: (i, 0)),
        out_shape=jax.ShapeDtypeStruct((N, 1), jnp.float32),
    )(h, agg, W1, b1, W2, b2, Wout, bout)


def kernel(x, edge_index, edge_attr,
           We0, be0, W10, b10, W20, b20,
           We1, be1, W11, b11, W21, b21,
           We2, be2, W12, b12, W22, b22,
           Wout, bout):
    W_stack = jnp.stack([We0, We1, We2])
    b_stack = jnp.stack([be0, be1, be2])
    e0, e1, e2 = _edge_linear(edge_attr, W_stack, b_stack)

    src2 = edge_index[0]
    dst2 = edge_index[1]

    h = x
    agg = _sc_message_pass(h, e0, src2, dst2)
    h = _mlp(h, agg, W10, b10, W20, b20)
    agg = _sc_message_pass(h, e1, src2, dst2)
    h = _mlp(h, agg, W11, b11, W21, b21)
    agg = _sc_message_pass(h, e2, src2, dst2)
    out = _mlp_head(h, agg, W12, b12, W22, b22, Wout, bout)
    return out.reshape(N)


# speedup vs baseline: 1.1542x; 1.1542x over previous
import functools

import jax
import jax.numpy as jnp
from jax import lax
from jax.experimental import pallas as pl
from jax.experimental.pallas import tpu as pltpu
from jax.experimental.pallas import tpu_sc as plsc

N = 10000
E = 320000
D = 128
ED = 16
H = 128

NC = 2
NS = 16
NW = NC * NS
C = 64
NCHUNK = E // C
KMAX = -(-NCHUNK // NW)
KP = ((KMAX + 2) // 3) * 3
RPT = 640
RPT_LAST = N - 15 * RPT


def _sc_message_pass(h, e, comb):
    mesh = plsc.VectorSubcoreMesh(core_axis_name="c", subcore_axis_name="s")

    @functools.partial(
        pl.kernel,
        mesh=mesh,
        out_type=jax.ShapeDtypeStruct((NC, N, D), jnp.float32),
        scratch_types=(
            [pltpu.VMEM((2 * C,), jnp.int32)] * 3
            + [pltpu.VMEM((C,), jnp.int32)] * 3
            + [pltpu.VMEM((C, D), jnp.float32)] * 3
            + [pltpu.VMEM((C, D), jnp.float32)] * 3
            + [pltpu.VMEM_SHARED((N, D), jnp.float32)]
            + [pltpu.SemaphoreType.DMA] * 6
        ),
    )
    def body(h_hbm, e_hbm, comb_hbm, out_hbm,
             iv0, iv1, iv2, dv0, dv1, dv2, rows0, rows1, rows2,
             ev0, ev1, ev2, acc, si0, si1, si2, sg0, sg1, sg2):
        bufs = ((iv0, dv0, rows0, ev0, si0, sg0),
                (iv1, dv1, rows1, ev1, si1, sg1),
                (iv2, dv2, rows2, ev2, si2, sg2))
        cid = lax.axis_index("c")
        sid = lax.axis_index("s")
        wid = cid * NS + sid

        r0 = sid * RPT

        @pl.when(sid < NS - 1)
        def _():
            pltpu.sync_copy(h_hbm.at[pl.ds(r0, RPT)], acc.at[pl.ds(r0, RPT)])

        @pl.when(sid == NS - 1)
        def _():
            pltpu.sync_copy(h_hbm.at[pl.ds(r0, RPT_LAST)],
                            acc.at[pl.ds(r0, RPT_LAST)])

        plsc.subcore_barrier()

        def start_idx(k, b):
            iv, dv, rv, ev, si, sg = bufs[b]
            chunk = wid + NW * k

            @pl.when(chunk < NCHUNK)
            def _():
                pltpu.async_copy(comb_hbm.at[pl.ds(chunk * 2 * C, 2 * C)],
                                 iv, si)

        def start_main(k, b):
            iv, dv, rv, ev, si, sg = bufs[b]
            chunk = wid + NW * k

            @pl.when(chunk < NCHUNK)
            def _():
                base = chunk * C
                pltpu.make_async_copy(
                    comb_hbm.at[pl.ds(chunk * 2 * C, 2 * C)], iv, si).wait()
                for j in range(C // 16):
                    s = pl.ds(j * 16, 16)
                    dv[s] = iv[pl.ds(C + j * 16, 16)]
                pltpu.async_copy(h_hbm.at[iv.at[pl.ds(0, C)]], rv, sg)
                pltpu.async_copy(e_hbm.at[pl.ds(base, C)], ev, sg)

        def finish(k, b):
            iv, dv, rv, ev, si, sg = bufs[b]
            chunk = wid + NW * k

            @pl.when(chunk < NCHUNK)
            def _():
                base = chunk * C
                pltpu.make_async_copy(h_hbm.at[iv.at[pl.ds(0, C)]], rv,
                                      sg).wait()
                pltpu.make_async_copy(e_hbm.at[pl.ds(base, C)], ev,
                                      sg).wait()

                def crow(r, c2):
                    for t in range(4):
                        for j in range(D // 16):
                            s = pl.ds(j * 16, 16)
                            rv[4 * r + t, s] = jnp.maximum(
                                rv[4 * r + t, s] + ev[4 * r + t, s], 0.0)
                    return c2

                lax.fori_loop(0, C // 4, crow, 0)
                pltpu.sync_copy(rv, acc.at[dv], add=True)

        start_idx(0, 0)
        start_idx(1, 1)
        start_idx(2, 2)
        start_main(0, 0)
        start_main(1, 1)

        def step(k3, carry):
            k = 3 * k3
            for t in range(3):
                start_main(k + t + 2, (t + 2) % 3)
                finish(k + t, t)
                start_idx(k + t + 3, t)
            return carry

        lax.fori_loop(0, KP // 3, step, 0)
        plsc.subcore_barrier()

        @pl.when(sid < NS - 1)
        def _():
            pltpu.sync_copy(acc.at[pl.ds(r0, RPT)],
                            out_hbm.at[cid, pl.ds(r0, RPT)])

        @pl.when(sid == NS - 1)
        def _():
            pltpu.sync_copy(acc.at[pl.ds(r0, RPT_LAST)],
                            out_hbm.at[cid, pl.ds(r0, RPT_LAST)])

    return body(h, e, comb)


BE = 4000


def _edge_linear(edge_attr, W_stack, b_stack):
    def body(ea_ref, w_ref, b_ref, o0, o1, o2):
        ea = ea_ref[...]
        outs = (o0, o1, o2)
        for l in range(3):
            outs[l][...] = jnp.dot(ea, w_ref[l],
                                   preferred_element_type=jnp.float32) + b_ref[l]

    return pl.pallas_call(
        body,
        grid=(E // BE,),
        in_specs=[
            pl.BlockSpec((BE, ED), lambda i: (i, 0)),
            pl.BlockSpec((3, ED, D), lambda i: (0, 0, 0)),
            pl.BlockSpec((3, D), lambda i: (0, 0)),
        ],
        out_specs=[pl.BlockSpec((BE, D), lambda i: (i, 0))] * 3,
        out_shape=[jax.ShapeDtypeStruct((E, D), jnp.float32)] * 3,
    )(edge_attr, W_stack, b_stack)


BN = 2000


def _mlp(h, agg, W1, b1, W2, b2):
    def body(h_ref, a_ref, w1, b1r, w2, b2r, out_ref):
        z = a_ref[0] + a_ref[1] - h_ref[...]
        z1 = jnp.maximum(jnp.dot(z, w1[...],
                                 preferred_element_type=jnp.float32) + b1r[...], 0.0)
        z2 = jnp.dot(z1, w2[...], preferred_element_type=jnp.float32) + b2r[...]
        out_ref[...] = jnp.maximum(z2, 0.0)

    return pl.pallas_call(
        body,
        grid=(N // BN,),
        in_specs=[
            pl.BlockSpec((BN, D), lambda i: (i, 0)),
            pl.BlockSpec((NC, BN, D), lambda i: (0, i, 0)),
            pl.BlockSpec((D, H), lambda i: (0, 0)),
            pl.BlockSpec((H,), lambda i: (0,)),
            pl.BlockSpec((H, H), lambda i: (0, 0)),
            pl.BlockSpec((H,), lambda i: (0,)),
        ],
        out_specs=pl.BlockSpec((BN, H), lambda i: (i, 0)),
        out_shape=jax.ShapeDtypeStruct((N, H), jnp.float32),
    )(h, agg, W1, b1, W2, b2)


def _mlp_head(h, agg, W1, b1, W2, b2, Wout, bout):
    def body(h_ref, a_ref, w1, b1r, w2, b2r, wo, bo, out_ref):
        z = a_ref[0] + a_ref[1] - h_ref[...]
        z1 = jnp.maximum(jnp.dot(z, w1[...],
                                 preferred_element_type=jnp.float32) + b1r[...], 0.0)
        z2 = jnp.dot(z1, w2[...], preferred_element_type=jnp.float32) + b2r[...]
        hf = jnp.maximum(z2, 0.0)
        logit = jnp.dot(hf, wo[...], preferred_element_type=jnp.float32) + bo[...]
        out_ref[...] = jax.nn.sigmoid(logit)

    return pl.pallas_call(
        body,
        grid=(N // BN,),
        in_specs=[
            pl.BlockSpec((BN, D), lambda i: (i, 0)),
            pl.BlockSpec((NC, BN, D), lambda i: (0, i, 0)),
            pl.BlockSpec((D, H), lambda i: (0, 0)),
            pl.BlockSpec((H,), lambda i: (0,)),
            pl.BlockSpec((H, H), lambda i: (0, 0)),
            pl.BlockSpec((H,), lambda i: (0,)),
            pl.BlockSpec((H, 1), lambda i: (0, 0)),
            pl.BlockSpec((1,), lambda i: (0,)),
        ],
        out_specs=pl.BlockSpec((BN, 1), lambda i: (i, 0)),
        out_shape=jax.ShapeDtypeStruct((N, 1), jnp.float32),
    )(h, agg, W1, b1, W2, b2, Wout, bout)


def kernel(x, edge_index, edge_attr,
           We0, be0, W10, b10, W20, b20,
           We1, be1, W11, b11, W21, b21,
           We2, be2, W12, b12, W22, b22,
           Wout, bout):
    W_stack = jnp.stack([We0, We1, We2])
    b_stack = jnp.stack([be0, be1, be2])
    e0, e1, e2 = _edge_linear(edge_attr, W_stack, b_stack)

    comb = jnp.concatenate(
        [edge_index[0].reshape(NCHUNK, C), edge_index[1].reshape(NCHUNK, C)],
        axis=1).reshape(-1)

    h = x
    agg = _sc_message_pass(h, e0, comb)
    h = _mlp(h, agg, W10, b10, W20, b20)
    agg = _sc_message_pass(h, e1, comb)
    h = _mlp(h, agg, W11, b11, W21, b21)
    agg = _sc_message_pass(h, e2, comb)
    out = _mlp_head(h, agg, W12, b12, W22, b22, Wout, bout)
    return out.reshape(N)


# speedup vs baseline: 1.2286x; 1.0645x over previous
import functools

import jax
import jax.numpy as jnp
from jax import lax
from jax.experimental import pallas as pl
from jax.experimental.pallas import tpu as pltpu
from jax.experimental.pallas import tpu_sc as plsc

N = 10000
E = 320000
D = 128
ED = 16
H = 128

NC = 2
NS = 16
HD = D // 2
NW = NC * NS
C = 64
NCHUNK = E // C
KMAX = -(-NCHUNK // NW)
KP = ((KMAX + 2) // 3) * 3
RPT = 640
RPT_LAST = N - 15 * RPT


def _sc_message_pass(h, ep, comb):
    mesh = plsc.VectorSubcoreMesh(core_axis_name="c", subcore_axis_name="s")

    @functools.partial(
        pl.kernel,
        mesh=mesh,
        out_type=jax.ShapeDtypeStruct((NC, N, D), jnp.float32),
        scratch_types=(
            [pltpu.VMEM((2 * C,), jnp.int32)] * 3
            + [pltpu.VMEM((C,), jnp.int32)] * 3
            + [pltpu.VMEM((C, D), jnp.float32)] * 3
            + [pltpu.VMEM((C, HD), jnp.int32)] * 3
            + [pltpu.VMEM_SHARED((N, D), jnp.float32)]
            + [pltpu.SemaphoreType.DMA] * 6
        ),
    )
    def body(h_hbm, e_hbm, comb_hbm, out_hbm,
             iv0, iv1, iv2, dv0, dv1, dv2, rows0, rows1, rows2,
             ev0, ev1, ev2, acc, si0, si1, si2, sg0, sg1, sg2):
        bufs = ((iv0, dv0, rows0, ev0, si0, sg0),
                (iv1, dv1, rows1, ev1, si1, sg1),
                (iv2, dv2, rows2, ev2, si2, sg2))
        cid = lax.axis_index("c")
        sid = lax.axis_index("s")
        wid = cid * NS + sid

        r0 = sid * RPT

        @pl.when(sid < NS - 1)
        def _():
            pltpu.sync_copy(h_hbm.at[pl.ds(r0, RPT)], acc.at[pl.ds(r0, RPT)])

        @pl.when(sid == NS - 1)
        def _():
            pltpu.sync_copy(h_hbm.at[pl.ds(r0, RPT_LAST)],
                            acc.at[pl.ds(r0, RPT_LAST)])

        plsc.subcore_barrier()

        def start_idx(k, b):
            iv, dv, rv, ev, si, sg = bufs[b]
            chunk = wid + NW * k

            @pl.when(chunk < NCHUNK)
            def _():
                pltpu.async_copy(comb_hbm.at[pl.ds(chunk * 2 * C, 2 * C)],
                                 iv, si)

        def start_main(k, b):
            iv, dv, rv, ev, si, sg = bufs[b]
            chunk = wid + NW * k

            @pl.when(chunk < NCHUNK)
            def _():
                base = chunk * C
                pltpu.make_async_copy(
                    comb_hbm.at[pl.ds(chunk * 2 * C, 2 * C)], iv, si).wait()
                for j in range(C // 16):
                    s = pl.ds(j * 16, 16)
                    dv[s] = iv[pl.ds(C + j * 16, 16)]
                pltpu.async_copy(h_hbm.at[iv.at[pl.ds(0, C)]], rv, sg)
                pltpu.async_copy(e_hbm.at[pl.ds(base, C)], ev, sg)

        def finish(k, b):
            iv, dv, rv, ev, si, sg = bufs[b]
            chunk = wid + NW * k

            @pl.when(chunk < NCHUNK)
            def _():
                base = chunk * C
                pltpu.make_async_copy(h_hbm.at[iv.at[pl.ds(0, C)]], rv,
                                      sg).wait()
                pltpu.make_async_copy(e_hbm.at[pl.ds(base, C)], ev,
                                      sg).wait()

                def crow(r, c2):
                    for t in range(2):
                        rr = 2 * r + t
                        for g in range(HD // 16):
                            sl = pl.ds(g * 16, 16)
                            sh = pl.ds(HD + g * 16, 16)
                            w = ev[rr, sl]
                            ea = lax.bitcast_convert_type(w << 16,
                                                          jnp.float32)
                            eb = lax.bitcast_convert_type(
                                w & jnp.int32(-65536), jnp.float32)
                            rv[rr, sl] = jnp.maximum(rv[rr, sl] + ea, 0.0)
                            rv[rr, sh] = jnp.maximum(rv[rr, sh] + eb, 0.0)
                    return c2

                lax.fori_loop(0, C // 2, crow, 0)
                pltpu.sync_copy(rv, acc.at[dv], add=True)

        start_idx(0, 0)
        start_idx(1, 1)
        start_idx(2, 2)
        start_main(0, 0)
        start_main(1, 1)

        def step(k3, carry):
            k = 3 * k3
            for t in range(3):
                start_main(k + t + 2, (t + 2) % 3)
                finish(k + t, t)
                start_idx(k + t + 3, t)
            return carry

        lax.fori_loop(0, KP // 3, step, 0)
        plsc.subcore_barrier()

        @pl.when(sid < NS - 1)
        def _():
            pltpu.sync_copy(acc.at[pl.ds(r0, RPT)],
                            out_hbm.at[cid, pl.ds(r0, RPT)])

        @pl.when(sid == NS - 1)
        def _():
            pltpu.sync_copy(acc.at[pl.ds(r0, RPT_LAST)],
                            out_hbm.at[cid, pl.ds(r0, RPT_LAST)])

    return body(h, ep, comb)


BE = 4000


def _pack_halves(z):
    a = lax.bitcast_convert_type(z[:, :HD].astype(jnp.bfloat16),
                                 jnp.uint16).astype(jnp.uint32)
    b = lax.bitcast_convert_type(z[:, HD:].astype(jnp.bfloat16),
                                 jnp.uint16).astype(jnp.uint32)
    return lax.bitcast_convert_type(a | (b << 16), jnp.int32)


def _edge_linear(edge_attr, W_stack, b_stack):
    def body(ea_ref, w_ref, b_ref, o0, o1, o2):
        ea = ea_ref[...]
        outs = (o0, o1, o2)
        for l in range(3):
            el = jnp.dot(ea, w_ref[l],
                         preferred_element_type=jnp.float32) + b_ref[l]
            outs[l][...] = _pack_halves(el)

    return pl.pallas_call(
        body,
        grid=(E // BE,),
        in_specs=[
            pl.BlockSpec((BE, ED), lambda i: (i, 0)),
            pl.BlockSpec((3, ED, D), lambda i: (0, 0, 0)),
            pl.BlockSpec((3, D), lambda i: (0, 0)),
        ],
        out_specs=[pl.BlockSpec((BE, HD), lambda i: (i, 0))] * 3,
        out_shape=[jax.ShapeDtypeStruct((E, HD), jnp.int32)] * 3,
    )(edge_attr, W_stack, b_stack)


BN = 2000


def _mlp(h, agg, W1, b1, W2, b2):
    def body(h_ref, a_ref, w1, b1r, w2, b2r, out_ref):
        z = a_ref[0] + a_ref[1] - h_ref[...]
        z1 = jnp.maximum(jnp.dot(z, w1[...],
                                 preferred_element_type=jnp.float32) + b1r[...], 0.0)
        z2 = jnp.dot(z1, w2[...], preferred_element_type=jnp.float32) + b2r[...]
        out_ref[...] = jnp.maximum(z2, 0.0)

    return pl.pallas_call(
        body,
        grid=(N // BN,),
        in_specs=[
            pl.BlockSpec((BN, D), lambda i: (i, 0)),
            pl.BlockSpec((NC, BN, D), lambda i: (0, i, 0)),
            pl.BlockSpec((D, H), lambda i: (0, 0)),
            pl.BlockSpec((H,), lambda i: (0,)),
            pl.BlockSpec((H, H), lambda i: (0, 0)),
            pl.BlockSpec((H,), lambda i: (0,)),
        ],
        out_specs=pl.BlockSpec((BN, H), lambda i: (i, 0)),
        out_shape=jax.ShapeDtypeStruct((N, H), jnp.float32),
    )(h, agg, W1, b1, W2, b2)


def _mlp_head(h, agg, W1, b1, W2, b2, Wout, bout):
    def body(h_ref, a_ref, w1, b1r, w2, b2r, wo, bo, out_ref):
        z = a_ref[0] + a_ref[1] - h_ref[...]
        z1 = jnp.maximum(jnp.dot(z, w1[...],
                                 preferred_element_type=jnp.float32) + b1r[...], 0.0)
        z2 = jnp.dot(z1, w2[...], preferred_element_type=jnp.float32) + b2r[...]
        hf = jnp.maximum(z2, 0.0)
        logit = jnp.dot(hf, wo[...], preferred_element_type=jnp.float32) + bo[...]
        out_ref[...] = jax.nn.sigmoid(logit)

    return pl.pallas_call(
        body,
        grid=(N // BN,),
        in_specs=[
            pl.BlockSpec((BN, D), lambda i: (i, 0)),
            pl.BlockSpec((NC, BN, D), lambda i: (0, i, 0)),
            pl.BlockSpec((D, H), lambda i: (0, 0)),
            pl.BlockSpec((H,), lambda i: (0,)),
            pl.BlockSpec((H, H), lambda i: (0, 0)),
            pl.BlockSpec((H,), lambda i: (0,)),
            pl.BlockSpec((H, 1), lambda i: (0, 0)),
            pl.BlockSpec((1,), lambda i: (0,)),
        ],
        out_specs=pl.BlockSpec((BN, 1), lambda i: (i, 0)),
        out_shape=jax.ShapeDtypeStruct((N, 1), jnp.float32),
    )(h, agg, W1, b1, W2, b2, Wout, bout)


def kernel(x, edge_index, edge_attr,
           We0, be0, W10, b10, W20, b20,
           We1, be1, W11, b11, W21, b21,
           We2, be2, W12, b12, W22, b22,
           Wout, bout):
    W_stack = jnp.stack([We0, We1, We2])
    b_stack = jnp.stack([be0, be1, be2])
    e0, e1, e2 = _edge_linear(edge_attr, W_stack, b_stack)

    comb = jnp.concatenate(
        [edge_index[0].reshape(NCHUNK, C), edge_index[1].reshape(NCHUNK, C)],
        axis=1).reshape(-1)

    h = x
    agg = _sc_message_pass(h, e0, comb)
    h = _mlp(h, agg, W10, b10, W20, b20)
    agg = _sc_message_pass(h, e1, comb)
    h = _mlp(h, agg, W11, b11, W21, b21)
    agg = _sc_message_pass(h, e2, comb)
    out = _mlp_head(h, agg, W12, b12, W22, b22, Wout, bout)
    return out.reshape(N)
